# baseline (speedup 1.0000x reference)
import functools

import numpy as np
import jax
import jax.numpy as jnp
from jax import lax
from jax.experimental import pallas as pl
from jax.experimental.pallas import tpu as pltpu
from jax.experimental.pallas import tpu_sc as plsc

_N = 10000
_E = 320000
_D = 128
_H = 128
_C = 40
_THETA = 0.5

_NPAD = 10240
_EPAD = 331776
_CPT = _EPAD // (16 * 128)

_NC = 2
_NS = 16
_NW = _NC * _NS
_AR = 4096
_RNG = _AR - 1

_rng = np.random.default_rng(0)
_neg_all = _rng.integers(0, _N, size=(2, _E))
_neg_all = _neg_all[:, _neg_all[0] < _neg_all[1]]
_NUM_NEG = _neg_all.shape[1]
_NEGPAD = ((_NUM_NEG + 4095) // 4096) * 4096
_NCH = _NEGPAD // (_NW * 128)
_NPT = _NCH * 128
_neg0_np = np.zeros(_NEGPAD, np.int32)
_neg0_np[:_NUM_NEG] = _neg_all[0]
_neg1_np = np.zeros(_NEGPAD, np.int32)
_neg1_np[:_NUM_NEG] = _neg_all[1]
_negm_np = np.zeros(_NEGPAD, np.float32)
_negm_np[:_NUM_NEG] = 1.0

_mesh = plsc.VectorSubcoreMesh(core_axis_name="c", subcore_axis_name="s")
_f32 = jnp.float32
_i32 = jnp.int32

_ECH = _EPAD // (_NW * 128)
_EPT = _ECH * 128


def _route(didx_v, sidx_v, lidx_v, gsrc_v, lo, use_src):
    for g in range(8):
        d16 = didx_v[pl.ds(g * 16, 16)]
        ok = (d16 >= lo) & (d16 < lo + _RNG)
        lidx_v[pl.ds(g * 16, 16)] = jnp.where(ok, d16 - lo, _RNG)
        if use_src:
            s16 = sidx_v[pl.ds(g * 16, 16)]
            gsrc_v[pl.ds(g * 16, 16)] = jnp.where(ok, s16, 0)


@functools.partial(
    pl.kernel,
    out_type=jax.ShapeDtypeStruct((2 * _NC, _AR, 128), _f32),
    mesh=_mesh,
    compiler_params=pltpu.CompilerParams(needs_layout_passes=False),
    scratch_types=[
        pltpu.VMEM((128,), _i32),
        pltpu.VMEM((128,), _i32),
        pltpu.VMEM((128, 128), _f32),
        pltpu.VMEM((128, 128), _f32),
        pltpu.VMEM_SHARED((_AR, 128), _f32),
    ],
)
def _deg_kernel(dst_hbm, zer_hbm, out_hbm, didx_v, lidx_v, ones_v, zer_v,
                acc_sh):
    cid = lax.axis_index("c")
    sid = lax.axis_index("s")

    def _fill(i, _):
        for j in range(8):
            ones_v[i, pl.ds(j * 16, 16)] = jnp.ones((16,), _f32)
        return 0

    lax.fori_loop(0, 128, _fill, 0)
    pltpu.sync_copy(zer_hbm.at[pl.ds(0, 128)], zer_v)

    for p in range(2):
        lo = (2 * p + cid) * _RNG
        for q in range(2):
            pltpu.sync_copy(zer_v, acc_sh.at[pl.ds(sid * 256 + q * 128, 128)])
        plsc.subcore_barrier()

        def _step(k, _):
            base = pl.multiple_of(sid * (_CPT * 128) + k * 128, 128)
            pltpu.sync_copy(dst_hbm.at[pl.ds(base, 128)], didx_v)
            _route(didx_v, didx_v, lidx_v, lidx_v, lo, False)
            pltpu.sync_copy(ones_v, acc_sh.at[lidx_v], add=True)
            return 0

        lax.fori_loop(0, _CPT, _step, 0)
        plsc.subcore_barrier()
        pltpu.sync_copy(acc_sh.at[pl.ds(sid * 256, 256)],
                        out_hbm.at[2 * p + cid, pl.ds(sid * 256, 256)])
        plsc.subcore_barrier()


_NBUF = 3


@functools.partial(
    pl.kernel,
    out_type=jax.ShapeDtypeStruct((2 * _NC, _AR, 128), _f32),
    mesh=_mesh,
    compiler_params=pltpu.CompilerParams(needs_layout_passes=False),
    scratch_types=(
        [pltpu.VMEM((_CPT * 128,), _i32), pltpu.VMEM((_CPT * 128,), _i32)]
        + [pltpu.VMEM((128,), _i32) for _ in range(2 * _NBUF)]
        + [pltpu.VMEM((128, 128), _f32) for _ in range(_NBUF)]
        + [pltpu.VMEM_SHARED((_AR, 128), _f32)]
        + [pltpu.SemaphoreType.DMA for _ in range(2 * _NBUF)]
    ),
)
def _agg_kernel(g_hbm, src_hbm, dst_hbm, zer_hbm, out_hbm,
                sidx_all, didx_all, gi0, gi1, gi2, li0, li1, li2,
                r0, r1, r2, acc_sh, gs0, gs1, gs2, ss0, ss1, ss2):
    cid = lax.axis_index("c")
    sid = lax.axis_index("s")
    gidx = [gi0, gi1, gi2]
    lidx = [li0, li1, li2]
    rows = [r0, r1, r2]
    gsem = [gs0, gs1, gs2]
    ssem = [ss0, ss1, ss2]
    nblk = _CPT // _NBUF

    for p in range(2):
        lo = (2 * p + cid) * _RNG
        for q in range(2):
            pltpu.sync_copy(zer_hbm.at[pl.ds(0, 128)],
                            acc_sh.at[pl.ds(sid * 256 + q * 128, 128)])
        base_all = pl.multiple_of(sid * (_CPT * 128), 128)
        pltpu.sync_copy(src_hbm.at[pl.ds(base_all, _CPT * 128)], sidx_all)
        pltpu.sync_copy(dst_hbm.at[pl.ds(base_all, _CPT * 128)], didx_all)
        plsc.subcore_barrier()

        def _block(blk, _):
            for b in range(_NBUF):
                koff = blk * _NBUF + b

                @pl.when(blk > 0)
                def _drain_sc():
                    pltpu.make_async_copy(rows[b], acc_sh.at[lidx[b]],
                                          ssem[b]).wait()

                for g in range(8):
                    off = koff * 128 + g * 16
                    d16 = didx_all[pl.ds(off, 16)]
                    s16 = sidx_all[pl.ds(off, 16)]
                    ok = (d16 >= lo) & (d16 < lo + _RNG)
                    lidx[b][pl.ds(g * 16, 16)] = jnp.where(ok, d16 - lo, _RNG)
                    gidx[b][pl.ds(g * 16, 16)] = jnp.where(ok, s16, 0)

                pltpu.async_copy(g_hbm.at[gidx[b]], rows[b], gsem[b])

                bp = (b - 1) % _NBUF
                if b == 0:
                    @pl.when(blk > 0)
                    def _fire_prev():
                        pltpu.make_async_copy(g_hbm.at[gidx[bp]], rows[bp],
                                              gsem[bp]).wait()
                        pltpu.async_copy(rows[bp], acc_sh.at[lidx[bp]],
                                         ssem[bp], add=True)
                else:
                    pltpu.make_async_copy(g_hbm.at[gidx[bp]], rows[bp],
                                          gsem[bp]).wait()
                    pltpu.async_copy(rows[bp], acc_sh.at[lidx[bp]],
                                     ssem[bp], add=True)
            return 0

        lax.fori_loop(0, nblk, _block, 0)
        pltpu.make_async_copy(g_hbm.at[gidx[2]], rows[2], gsem[2]).wait()
        pltpu.async_copy(rows[2], acc_sh.at[lidx[2]], ssem[2], add=True)
        for b in range(_NBUF):
            pltpu.make_async_copy(rows[b], acc_sh.at[lidx[b]], ssem[b]).wait()
        plsc.subcore_barrier()
        pltpu.sync_copy(acc_sh.at[pl.ds(sid * 256, 256)],
                        out_hbm.at[2 * p + cid, pl.ds(sid * 256, 256)])
        plsc.subcore_barrier()


@functools.partial(
    pl.kernel,
    out_type=jax.ShapeDtypeStruct((_NW, 128), _f32),
    mesh=_mesh,
    compiler_params=pltpu.CompilerParams(needs_layout_passes=False),
    scratch_types=[
        pltpu.VMEM((128,), _i32),
        pltpu.VMEM((128,), _i32),
        pltpu.VMEM((128,), _i32),
        pltpu.VMEM((128,), _f32),
        pltpu.VMEM((128,), _f32),
        pltpu.VMEM((128,), _f32),
        pltpu.VMEM((128, _H), _f32),
        pltpu.VMEM((128, _H), _f32),
        pltpu.SemaphoreType.DMA,
    ],
)
def _loss_kernel(rep_hbm, src_hbm, dst_hbm, simf_hbm, n0_hbm, n1_hbm, nm_hbm,
                 out_hbm, sidx_v, didx_v, qidx_v, fsim_v, nm_v, rowbuf,
                 rows_a, rows_b, sem):
    cid = lax.axis_index("c")
    sid = lax.axis_index("s")
    wid = cid * _NS + sid
    il = lax.iota(_i32, 16)

    def _dots16(g):
        rbase = il + g * 16

        def _dstep(dd, acc):
            cidx = jnp.zeros((16,), _i32) + dd
            ca = plsc.load_gather(rows_a, [rbase, cidx])
            cb = plsc.load_gather(rows_b, [rbase, cidx])
            return acc + ca * cb

        return lax.fori_loop(0, _H, _dstep, jnp.zeros((16,), _f32), unroll=8)

    def _pos_chunk(k, carry):
        ps, pc = carry
        base = pl.multiple_of(wid * _EPT + k * 128, 128)
        pltpu.sync_copy(src_hbm.at[pl.ds(base, 128)], sidx_v)
        pltpu.sync_copy(dst_hbm.at[pl.ds(base, 128)], didx_v)
        pltpu.async_copy(rep_hbm.at[sidx_v], rows_a, sem).wait()
        pltpu.async_copy(rep_hbm.at[didx_v], rows_b, sem).wait()
        for j in range(8):
            s = sidx_v[pl.ds(j * 16, 16)]
            d = didx_v[pl.ds(j * 16, 16)]
            qidx_v[pl.ds(j * 16, 16)] = s * _N + d
        pltpu.async_copy(simf_hbm.at[qidx_v], fsim_v, sem).wait()
        for g in range(8):
            s16 = sidx_v[pl.ds(g * 16, 16)]
            d16 = didx_v[pl.ds(g * 16, 16)]
            fs16 = fsim_v[pl.ds(g * 16, 16)]
            w = jnp.maximum(_dots16(g), 0.0)
            pv = fs16 * _THETA + w * (1.0 - _THETA)
            valid = s16 < d16
            ps = ps + jnp.where(valid, (pv - 1.0) * (pv - 1.0), 0.0)
            pc = pc + jnp.where(valid, 1.0, 0.0)
        return (ps, pc)

    zz = jnp.zeros((16,), _f32)
    psv, pcv = lax.fori_loop(0, _ECH, _pos_chunk, (zz, zz))

    def _neg_chunk(k, carry):
        ns = carry
        base = pl.multiple_of(wid * _NPT + k * 128, 128)
        pltpu.sync_copy(n0_hbm.at[pl.ds(base, 128)], sidx_v)
        pltpu.sync_copy(n1_hbm.at[pl.ds(base, 128)], didx_v)
        pltpu.sync_copy(nm_hbm.at[pl.ds(base, 128)], nm_v)
        pltpu.async_copy(rep_hbm.at[sidx_v], rows_a, sem).wait()
        pltpu.async_copy(rep_hbm.at[didx_v], rows_b, sem).wait()
        for g in range(8):
            nm16 = nm_v[pl.ds(g * 16, 16)]
            w = jnp.maximum(_dots16(g), 0.0)
            ns = ns + nm16 * w * w
        return ns

    nsv = lax.fori_loop(0, _NCH, _neg_chunk, zz)
    psum = jnp.sum(psv)
    pcnt = jnp.sum(pcv)
    nsum = jnp.sum(nsv)

    for j in range(8):
        rowbuf[pl.ds(j * 16, 16)] = jnp.zeros((16,), _f32)
    vec = (jnp.where(il == 0, psum, 0.0) + jnp.where(il == 1, pcnt, 0.0)
           + jnp.where(il == 2, nsum, 0.0))
    rowbuf[pl.ds(0, 16)] = vec
    pltpu.sync_copy(rowbuf, out_hbm.at[wid])


_BLK = 512
_GRID = _NPAD // _BLK


def _mm1_body(deg_ref, feat_ref, w1_ref, dis_ref, g1_ref):
    i = pl.program_id(0)
    dis = lax.rsqrt(deg_ref[pl.ds(i * _BLK, _BLK)])
    dis_ref[pl.ds(i * _BLK, _BLK)] = dis
    h = lax.dot_general(feat_ref[...], w1_ref[...], (((1,), (1,)), ((), ())),
                        preferred_element_type=_f32)
    g1_ref[...] = dis[:, None] * h


def _mm1(deg, feat_p, W1):
    return pl.pallas_call(
        _mm1_body,
        grid=(_GRID,),
        in_specs=[
            pl.BlockSpec((_NPAD,), lambda i: (0,)),
            pl.BlockSpec((_BLK, _D), lambda i: (i, 0)),
            pl.BlockSpec((_H, _D), lambda i: (0, 0)),
        ],
        out_specs=[
            pl.BlockSpec((_NPAD,), lambda i: (0,)),
            pl.BlockSpec((_BLK, _H), lambda i: (i, 0)),
        ],
        out_shape=[
            jax.ShapeDtypeStruct((_NPAD,), _f32),
            jax.ShapeDtypeStruct((_NPAD, _H), _f32),
        ],
    )(deg, feat_p, W1)


def _mid_body(dis_ref, b1_ref, agg_ref, g1_ref, w2_ref, g2_ref):
    i = pl.program_id(0)
    dis = dis_ref[pl.ds(i * _BLK, _BLK)]
    agg = agg_ref[...] + g1_ref[...]
    x1 = jnp.maximum(dis[:, None] * agg + b1_ref[...][None, :], 0.0)
    h2 = lax.dot_general(x1, w2_ref[...], (((1,), (1,)), ((), ())),
                         preferred_element_type=_f32)
    g2_ref[...] = dis[:, None] * h2


def _mid(dis, b1, agg1, g1, W2):
    return pl.pallas_call(
        _mid_body,
        grid=(_GRID,),
        in_specs=[
            pl.BlockSpec((_NPAD,), lambda i: (0,)),
            pl.BlockSpec((_H,), lambda i: (0,)),
            pl.BlockSpec((_BLK, _H), lambda i: (i, 0)),
            pl.BlockSpec((_BLK, _H), lambda i: (i, 0)),
            pl.BlockSpec((_H, _H), lambda i: (0, 0)),
        ],
        out_specs=pl.BlockSpec((_BLK, _H), lambda i: (i, 0)),
        out_shape=jax.ShapeDtypeStruct((_NPAD, _H), _f32),
    )(dis, b1, agg1, g1, W2)


def _head_body(dis_ref, b2_ref, by_ref, agg_ref, g2_ref, wy_ref,
               rep_ref, y_ref):
    i = pl.program_id(0)
    dis = dis_ref[pl.ds(i * _BLK, _BLK)]
    agg = agg_ref[...] + g2_ref[...]
    x2 = dis[:, None] * agg + b2_ref[...][None, :]
    n1 = jnp.sqrt(jnp.sum(x2 * x2, axis=1, keepdims=True))
    r1 = x2 / jnp.maximum(n1, 1e-12)
    n2 = jnp.sqrt(jnp.sum(r1 * r1, axis=1, keepdims=True))
    rep = r1 / jnp.maximum(n2, 1e-12)
    rep_ref[...] = rep
    y_ref[...] = lax.dot_general(rep, wy_ref[...], (((1,), (1,)), ((), ())),
                                 preferred_element_type=_f32) + by_ref[...][None, :]


def _head(dis, b2, by_p, agg2, g2, Wy_p):
    return pl.pallas_call(
        _head_body,
        grid=(_GRID,),
        in_specs=[
            pl.BlockSpec((_NPAD,), lambda i: (0,)),
            pl.BlockSpec((_H,), lambda i: (0,)),
            pl.BlockSpec((128,), lambda i: (0,)),
            pl.BlockSpec((_BLK, _H), lambda i: (i, 0)),
            pl.BlockSpec((_BLK, _H), lambda i: (i, 0)),
            pl.BlockSpec((128, _H), lambda i: (0, 0)),
        ],
        out_specs=[
            pl.BlockSpec((_BLK, _H), lambda i: (i, 0)),
            pl.BlockSpec((_BLK, 128), lambda i: (i, 0)),
        ],
        out_shape=[
            jax.ShapeDtypeStruct((_NPAD, _H), _f32),
            jax.ShapeDtypeStruct((_NPAD, 128), _f32),
        ],
    )(dis, b2, by_p, agg2, g2, Wy_p)


def _fin_body(parts_ref, out_ref):
    parts = parts_ref[...]
    col = lax.broadcasted_iota(_i32, (_NW, 128), 1)
    ps = jnp.sum(jnp.where(col == 0, parts, 0.0))
    pc = jnp.sum(jnp.where(col == 1, parts, 0.0))
    ns = jnp.sum(jnp.where(col == 2, parts, 0.0))
    rec = (ns + ps) * float(_N) / (float(_NUM_NEG) + pc)
    out_ref[...] = jnp.full((8, 128), rec, _f32)


def _fin(parts):
    return pl.pallas_call(
        _fin_body,
        grid=(1,),
        in_specs=[pl.BlockSpec((_NW, 128), lambda i: (0, 0))],
        out_specs=pl.BlockSpec((8, 128), lambda i: (0, 0)),
        out_shape=jax.ShapeDtypeStruct((8, 128), _f32),
    )(parts)


def _assemble(outp):
    return jnp.concatenate(
        [outp[0, :_RNG], outp[1, :_RNG], outp[2, :_NPAD - 2 * _RNG]], axis=0)


def kernel(edge_index, features, sim, W1, b1, W2, b2, Wy, by):
    src = edge_index[0].astype(_i32)
    dst = edge_index[1].astype(_i32)
    padz = jnp.zeros(_EPAD - _E, _i32)
    src_p = jnp.concatenate([src, padz])
    dst_l = jnp.concatenate([dst, padz])
    dst_a = jnp.concatenate([dst, jnp.full(_EPAD - _E, _N, _i32)])

    feat_p = jnp.pad(features, ((0, _NPAD - _N), (0, 0)))
    Wy_p = jnp.pad(Wy, ((0, 128 - _C), (0, 0)))
    by_p = jnp.pad(by, (0, 128 - _C))
    sim_flat = sim.reshape(-1)
    zer128 = jnp.zeros((128, 128), _f32)

    neg0 = jnp.asarray(_neg0_np)
    neg1 = jnp.asarray(_neg1_np)
    negm = jnp.asarray(_negm_np)

    degp = _deg_kernel(dst_a, zer128)
    deg = _assemble(degp)[:, 0] + 1.0

    dis, g1 = _mm1(deg, feat_p, W1)
    agg1 = _assemble(_agg_kernel(g1, src_p, dst_a, zer128))
    g2 = _mid(dis, b1, agg1, g1, W2)
    agg2 = _assemble(_agg_kernel(g2, src_p, dst_a, zer128))
    rep_p, y_p = _head(dis, b2, by_p, agg2, g2, Wy_p)

    parts = _loss_kernel(rep_p, src_p, dst_l, sim_flat, neg0, neg1, negm)
    recb = _fin(parts)

    return rep_p[:_N], recb[0, 0], y_p[:_N, :_C]


# speedup vs baseline: 8.9261x; 8.9261x over previous
import functools

import numpy as np
import jax
import jax.numpy as jnp
from jax import lax
from jax.experimental import pallas as pl
from jax.experimental.pallas import tpu as pltpu
from jax.experimental.pallas import tpu_sc as plsc

_N = 10000
_E = 320000
_D = 128
_H = 128
_C = 40
_THETA = 0.5

_NPAD = 10240
_EPAD = 331776
_CPT = _EPAD // (16 * 128)

_NC = 2
_NS = 16
_NW = _NC * _NS
_AR = 4096
_RNG = _AR - 1

_rng = np.random.default_rng(0)
_neg_all = _rng.integers(0, _N, size=(2, _E))
_neg_all = _neg_all[:, _neg_all[0] < _neg_all[1]]
_NUM_NEG = _neg_all.shape[1]
_NEGPAD = ((_NUM_NEG + 4095) // 4096) * 4096
_NCH = _NEGPAD // (_NW * 128)
_NPT = _NCH * 128
_neg0_np = np.zeros(_NEGPAD, np.int32)
_neg0_np[:_NUM_NEG] = _neg_all[0]
_neg1_np = np.zeros(_NEGPAD, np.int32)
_neg1_np[:_NUM_NEG] = _neg_all[1]
_negm_np = np.zeros(_NEGPAD, np.float32)
_negm_np[:_NUM_NEG] = 1.0

_mesh = plsc.VectorSubcoreMesh(core_axis_name="c", subcore_axis_name="s")
_f32 = jnp.float32
_i32 = jnp.int32

_ECH = _EPAD // (_NW * 128)
_EPT = _ECH * 128


def _route(didx_v, sidx_v, lidx_v, gsrc_v, lo, use_src):
    for g in range(8):
        d16 = didx_v[pl.ds(g * 16, 16)]
        ok = (d16 >= lo) & (d16 < lo + _RNG)
        lidx_v[pl.ds(g * 16, 16)] = jnp.where(ok, d16 - lo, _RNG)
        if use_src:
            s16 = sidx_v[pl.ds(g * 16, 16)]
            gsrc_v[pl.ds(g * 16, 16)] = jnp.where(ok, s16, 0)


@functools.partial(
    pl.kernel,
    out_type=jax.ShapeDtypeStruct((2 * _NC, _AR, 128), _f32),
    mesh=_mesh,
    compiler_params=pltpu.CompilerParams(needs_layout_passes=False),
    scratch_types=[
        pltpu.VMEM((128,), _i32),
        pltpu.VMEM((128,), _i32),
        pltpu.VMEM((128, 128), _f32),
        pltpu.VMEM((128, 128), _f32),
        pltpu.VMEM_SHARED((_AR, 128), _f32),
    ],
)
def _deg_kernel(dst_hbm, zer_hbm, out_hbm, didx_v, lidx_v, ones_v, zer_v,
                acc_sh):
    cid = lax.axis_index("c")
    sid = lax.axis_index("s")

    def _fill(i, _):
        for j in range(8):
            ones_v[i, pl.ds(j * 16, 16)] = jnp.ones((16,), _f32)
        return 0

    lax.fori_loop(0, 128, _fill, 0)
    pltpu.sync_copy(zer_hbm.at[pl.ds(0, 128)], zer_v)

    for p in range(2):
        lo = (2 * p + cid) * _RNG
        for q in range(2):
            pltpu.sync_copy(zer_v, acc_sh.at[pl.ds(sid * 256 + q * 128, 128)])
        plsc.subcore_barrier()

        def _step(k, _):
            base = pl.multiple_of(sid * (_CPT * 128) + k * 128, 128)
            pltpu.sync_copy(dst_hbm.at[pl.ds(base, 128)], didx_v)
            _route(didx_v, didx_v, lidx_v, lidx_v, lo, False)
            pltpu.sync_copy(ones_v, acc_sh.at[lidx_v], add=True)
            return 0

        lax.fori_loop(0, _CPT, _step, 0)
        plsc.subcore_barrier()
        pltpu.sync_copy(acc_sh.at[pl.ds(sid * 256, 256)],
                        out_hbm.at[2 * p + cid, pl.ds(sid * 256, 256)])
        plsc.subcore_barrier()


_NBUF = 3


@functools.partial(
    pl.kernel,
    out_type=jax.ShapeDtypeStruct((2 * _NC, _AR, 128), _f32),
    mesh=_mesh,
    compiler_params=pltpu.CompilerParams(needs_layout_passes=False),
    scratch_types=(
        [pltpu.VMEM((_CPT * 128,), _i32), pltpu.VMEM((_CPT * 128,), _i32)]
        + [pltpu.VMEM((128,), _i32) for _ in range(2 * _NBUF)]
        + [pltpu.VMEM((128, 128), _f32) for _ in range(_NBUF)]
        + [pltpu.VMEM_SHARED((_AR, 128), _f32)]
        + [pltpu.SemaphoreType.DMA for _ in range(2 * _NBUF)]
    ),
)
def _agg_kernel(g_hbm, src_hbm, dst_hbm, zer_hbm, out_hbm,
                sidx_all, didx_all, gi0, gi1, gi2, li0, li1, li2,
                r0, r1, r2, acc_sh, gs0, gs1, gs2, ss0, ss1, ss2):
    cid = lax.axis_index("c")
    sid = lax.axis_index("s")
    gidx = [gi0, gi1, gi2]
    lidx = [li0, li1, li2]
    rows = [r0, r1, r2]
    gsem = [gs0, gs1, gs2]
    ssem = [ss0, ss1, ss2]
    nblk = _CPT // _NBUF

    for p in range(2):
        lo = (2 * p + cid) * _RNG
        for q in range(2):
            pltpu.sync_copy(zer_hbm.at[pl.ds(0, 128)],
                            acc_sh.at[pl.ds(sid * 256 + q * 128, 128)])
        base_all = pl.multiple_of(sid * (_CPT * 128), 128)
        pltpu.sync_copy(src_hbm.at[pl.ds(base_all, _CPT * 128)], sidx_all)
        pltpu.sync_copy(dst_hbm.at[pl.ds(base_all, _CPT * 128)], didx_all)
        plsc.subcore_barrier()

        def _block(blk, _):
            for b in range(_NBUF):
                koff = blk * _NBUF + b

                @pl.when(blk > 0)
                def _drain_sc():
                    pltpu.make_async_copy(rows[b], acc_sh.at[lidx[b]],
                                          ssem[b]).wait()

                for g in range(8):
                    off = koff * 128 + g * 16
                    d16 = didx_all[pl.ds(off, 16)]
                    s16 = sidx_all[pl.ds(off, 16)]
                    ok = (d16 >= lo) & (d16 < lo + _RNG)
                    lidx[b][pl.ds(g * 16, 16)] = jnp.where(ok, d16 - lo, _RNG)
                    gidx[b][pl.ds(g * 16, 16)] = s16

                pltpu.async_copy(g_hbm.at[gidx[b]], rows[b], gsem[b])

                bp = (b - 1) % _NBUF
                if b == 0:
                    @pl.when(blk > 0)
                    def _fire_prev():
                        pltpu.make_async_copy(g_hbm.at[gidx[bp]], rows[bp],
                                              gsem[bp]).wait()
                        pltpu.async_copy(rows[bp], acc_sh.at[lidx[bp]],
                                         ssem[bp], add=True)
                else:
                    pltpu.make_async_copy(g_hbm.at[gidx[bp]], rows[bp],
                                          gsem[bp]).wait()
                    pltpu.async_copy(rows[bp], acc_sh.at[lidx[bp]],
                                     ssem[bp], add=True)
            return 0

        lax.fori_loop(0, nblk, _block, 0)
        pltpu.make_async_copy(g_hbm.at[gidx[2]], rows[2], gsem[2]).wait()
        pltpu.async_copy(rows[2], acc_sh.at[lidx[2]], ssem[2], add=True)
        for b in range(_NBUF):
            pltpu.make_async_copy(rows[b], acc_sh.at[lidx[b]], ssem[b]).wait()
        plsc.subcore_barrier()
        pltpu.sync_copy(acc_sh.at[pl.ds(sid * 256, 256)],
                        out_hbm.at[2 * p + cid, pl.ds(sid * 256, 256)])
        plsc.subcore_barrier()


@functools.partial(
    pl.kernel,
    out_type=jax.ShapeDtypeStruct((_NW, 128), _f32),
    mesh=_mesh,
    compiler_params=pltpu.CompilerParams(needs_layout_passes=False),
    scratch_types=[
        pltpu.VMEM((128,), _i32),
        pltpu.VMEM((128,), _i32),
        pltpu.VMEM((128,), _i32),
        pltpu.VMEM((128,), _f32),
        pltpu.VMEM((128,), _f32),
        pltpu.VMEM((128,), _f32),
        pltpu.VMEM((128, _H), _f32),
        pltpu.VMEM((128, _H), _f32),
        pltpu.SemaphoreType.DMA,
    ],
)
def _loss_kernel(rep_hbm, src_hbm, dst_hbm, simf_hbm, n0_hbm, n1_hbm, nm_hbm,
                 out_hbm, sidx_v, didx_v, qidx_v, fsim_v, nm_v, rowbuf,
                 rows_a, rows_b, sem):
    cid = lax.axis_index("c")
    sid = lax.axis_index("s")
    wid = cid * _NS + sid
    il = lax.iota(_i32, 16)

    def _dots16(g):
        rbase = il + g * 16

        def _dstep(dd, acc):
            cidx = jnp.zeros((16,), _i32) + dd
            ca = plsc.load_gather(rows_a, [rbase, cidx])
            cb = plsc.load_gather(rows_b, [rbase, cidx])
            return acc + ca * cb

        return lax.fori_loop(0, _H, _dstep, jnp.zeros((16,), _f32), unroll=8)

    def _pos_chunk(k, carry):
        ps, pc = carry
        base = pl.multiple_of(wid * _EPT + k * 128, 128)
        pltpu.sync_copy(src_hbm.at[pl.ds(base, 128)], sidx_v)
        pltpu.sync_copy(dst_hbm.at[pl.ds(base, 128)], didx_v)
        pltpu.async_copy(rep_hbm.at[sidx_v], rows_a, sem).wait()
        pltpu.async_copy(rep_hbm.at[didx_v], rows_b, sem).wait()
        for j in range(8):
            s = sidx_v[pl.ds(j * 16, 16)]
            d = didx_v[pl.ds(j * 16, 16)]
            qidx_v[pl.ds(j * 16, 16)] = s * _N + d
        pltpu.async_copy(simf_hbm.at[qidx_v], fsim_v, sem).wait()
        for g in range(8):
            s16 = sidx_v[pl.ds(g * 16, 16)]
            d16 = didx_v[pl.ds(g * 16, 16)]
            fs16 = fsim_v[pl.ds(g * 16, 16)]
            w = jnp.maximum(_dots16(g), 0.0)
            pv = fs16 * _THETA + w * (1.0 - _THETA)
            valid = s16 < d16
            ps = ps + jnp.where(valid, (pv - 1.0) * (pv - 1.0), 0.0)
            pc = pc + jnp.where(valid, 1.0, 0.0)
        return (ps, pc)

    zz = jnp.zeros((16,), _f32)
    psv, pcv = lax.fori_loop(0, _ECH, _pos_chunk, (zz, zz))

    def _neg_chunk(k, carry):
        ns = carry
        base = pl.multiple_of(wid * _NPT + k * 128, 128)
        pltpu.sync_copy(n0_hbm.at[pl.ds(base, 128)], sidx_v)
        pltpu.sync_copy(n1_hbm.at[pl.ds(base, 128)], didx_v)
        pltpu.sync_copy(nm_hbm.at[pl.ds(base, 128)], nm_v)
        pltpu.async_copy(rep_hbm.at[sidx_v], rows_a, sem).wait()
        pltpu.async_copy(rep_hbm.at[didx_v], rows_b, sem).wait()
        for g in range(8):
            nm16 = nm_v[pl.ds(g * 16, 16)]
            w = jnp.maximum(_dots16(g), 0.0)
            ns = ns + nm16 * w * w
        return ns

    nsv = lax.fori_loop(0, _NCH, _neg_chunk, zz)
    psum = jnp.sum(psv)
    pcnt = jnp.sum(pcv)
    nsum = jnp.sum(nsv)

    for j in range(8):
        rowbuf[pl.ds(j * 16, 16)] = jnp.zeros((16,), _f32)
    vec = (jnp.where(il == 0, psum, 0.0) + jnp.where(il == 1, pcnt, 0.0)
           + jnp.where(il == 2, nsum, 0.0))
    rowbuf[pl.ds(0, 16)] = vec
    pltpu.sync_copy(rowbuf, out_hbm.at[wid])


_BLK = 512
_GRID = _NPAD // _BLK


def _mm1_body(deg_ref, feat_ref, w1_ref, dis_ref, g1_ref):
    i = pl.program_id(0)
    dis = lax.rsqrt(deg_ref[pl.ds(i * _BLK, _BLK)])
    dis_ref[pl.ds(i * _BLK, _BLK)] = dis
    h = lax.dot_general(feat_ref[...], w1_ref[...], (((1,), (1,)), ((), ())),
                        preferred_element_type=_f32)
    g1_ref[...] = dis[:, None] * h


def _mm1(deg, feat_p, W1):
    return pl.pallas_call(
        _mm1_body,
        grid=(_GRID,),
        in_specs=[
            pl.BlockSpec((_NPAD,), lambda i: (0,)),
            pl.BlockSpec((_BLK, _D), lambda i: (i, 0)),
            pl.BlockSpec((_H, _D), lambda i: (0, 0)),
        ],
        out_specs=[
            pl.BlockSpec((_NPAD,), lambda i: (0,)),
            pl.BlockSpec((_BLK, _H), lambda i: (i, 0)),
        ],
        out_shape=[
            jax.ShapeDtypeStruct((_NPAD,), _f32),
            jax.ShapeDtypeStruct((_NPAD, _H), _f32),
        ],
    )(deg, feat_p, W1)


def _mid_body(dis_ref, b1_ref, agg_ref, g1_ref, w2_ref, g2_ref):
    i = pl.program_id(0)
    dis = dis_ref[pl.ds(i * _BLK, _BLK)]
    agg = agg_ref[...] + g1_ref[...]
    x1 = jnp.maximum(dis[:, None] * agg + b1_ref[...][None, :], 0.0)
    h2 = lax.dot_general(x1, w2_ref[...], (((1,), (1,)), ((), ())),
                         preferred_element_type=_f32)
    g2_ref[...] = dis[:, None] * h2


def _mid(dis, b1, agg1, g1, W2):
    return pl.pallas_call(
        _mid_body,
        grid=(_GRID,),
        in_specs=[
            pl.BlockSpec((_NPAD,), lambda i: (0,)),
            pl.BlockSpec((_H,), lambda i: (0,)),
            pl.BlockSpec((_BLK, _H), lambda i: (i, 0)),
            pl.BlockSpec((_BLK, _H), lambda i: (i, 0)),
            pl.BlockSpec((_H, _H), lambda i: (0, 0)),
        ],
        out_specs=pl.BlockSpec((_BLK, _H), lambda i: (i, 0)),
        out_shape=jax.ShapeDtypeStruct((_NPAD, _H), _f32),
    )(dis, b1, agg1, g1, W2)


def _head_body(dis_ref, b2_ref, by_ref, agg_ref, g2_ref, wy_ref,
               rep_ref, y_ref):
    i = pl.program_id(0)
    dis = dis_ref[pl.ds(i * _BLK, _BLK)]
    agg = agg_ref[...] + g2_ref[...]
    x2 = dis[:, None] * agg + b2_ref[...][None, :]
    n1 = jnp.sqrt(jnp.sum(x2 * x2, axis=1, keepdims=True))
    r1 = x2 / jnp.maximum(n1, 1e-12)
    n2 = jnp.sqrt(jnp.sum(r1 * r1, axis=1, keepdims=True))
    rep = r1 / jnp.maximum(n2, 1e-12)
    rep_ref[...] = rep
    y_ref[...] = lax.dot_general(rep, wy_ref[...], (((1,), (1,)), ((), ())),
                                 preferred_element_type=_f32) + by_ref[...][None, :]


def _head(dis, b2, by_p, agg2, g2, Wy_p):
    return pl.pallas_call(
        _head_body,
        grid=(_GRID,),
        in_specs=[
            pl.BlockSpec((_NPAD,), lambda i: (0,)),
            pl.BlockSpec((_H,), lambda i: (0,)),
            pl.BlockSpec((128,), lambda i: (0,)),
            pl.BlockSpec((_BLK, _H), lambda i: (i, 0)),
            pl.BlockSpec((_BLK, _H), lambda i: (i, 0)),
            pl.BlockSpec((128, _H), lambda i: (0, 0)),
        ],
        out_specs=[
            pl.BlockSpec((_BLK, _H), lambda i: (i, 0)),
            pl.BlockSpec((_BLK, 128), lambda i: (i, 0)),
        ],
        out_shape=[
            jax.ShapeDtypeStruct((_NPAD, _H), _f32),
            jax.ShapeDtypeStruct((_NPAD, 128), _f32),
        ],
    )(dis, b2, by_p, agg2, g2, Wy_p)


def _fin_body(parts_ref, out_ref):
    parts = parts_ref[...]
    col = lax.broadcasted_iota(_i32, (_NW, 128), 1)
    ps = jnp.sum(jnp.where(col == 0, parts, 0.0))
    pc = jnp.sum(jnp.where(col == 1, parts, 0.0))
    ns = jnp.sum(jnp.where(col == 2, parts, 0.0))
    rec = (ns + ps) * float(_N) / (float(_NUM_NEG) + pc)
    out_ref[...] = jnp.full((8, 128), rec, _f32)


def _fin(parts):
    return pl.pallas_call(
        _fin_body,
        grid=(1,),
        in_specs=[pl.BlockSpec((_NW, 128), lambda i: (0, 0))],
        out_specs=pl.BlockSpec((8, 128), lambda i: (0, 0)),
        out_shape=jax.ShapeDtypeStruct((8, 128), _f32),
    )(parts)


def _assemble(outp):
    return jnp.concatenate(
        [outp[0, :_RNG], outp[1, :_RNG], outp[2, :_NPAD - 2 * _RNG]], axis=0)


def kernel(edge_index, features, sim, W1, b1, W2, b2, Wy, by):
    src = edge_index[0].astype(_i32)
    dst = edge_index[1].astype(_i32)
    padz = jnp.zeros(_EPAD - _E, _i32)
    src_p = jnp.concatenate([src, padz])
    dst_l = jnp.concatenate([dst, padz])
    dst_a = jnp.concatenate([dst, jnp.full(_EPAD - _E, _N, _i32)])

    feat_p = jnp.pad(features, ((0, _NPAD - _N), (0, 0)))
    Wy_p = jnp.pad(Wy, ((0, 128 - _C), (0, 0)))
    by_p = jnp.pad(by, (0, 128 - _C))
    sim_flat = sim.reshape(-1)
    zer128 = jnp.zeros((128, 128), _f32)

    neg0 = jnp.asarray(_neg0_np)
    neg1 = jnp.asarray(_neg1_np)
    negm = jnp.asarray(_negm_np)

    degp = _deg_kernel(dst_a, zer128)
    deg = _assemble(degp)[:, 0] + 1.0

    dis, g1 = _mm1(deg, feat_p, W1)
    agg1 = _assemble(_agg_kernel(g1, src_p, dst_a, zer128))
    g2 = _mid(dis, b1, agg1, g1, W2)
    agg2 = _assemble(_agg_kernel(g2, src_p, dst_a, zer128))
    rep_p, y_p = _head(dis, b2, by_p, agg2, g2, Wy_p)

    parts = _loss_kernel(rep_p, src_p, dst_l, sim_flat, neg0, neg1, negm)
    recb = _fin(parts)

    return rep_p[:_N], recb[0, 0], y_p[:_N, :_C]


# speedup vs baseline: 9.0547x; 1.0144x over previous
import functools

import numpy as np
import jax
import jax.numpy as jnp
from jax import lax
from jax.experimental import pallas as pl
from jax.experimental.pallas import tpu as pltpu
from jax.experimental.pallas import tpu_sc as plsc

_N = 10000
_E = 320000
_D = 128
_H = 128
_C = 40
_THETA = 0.5

_NPAD = 10240
_EPAD = 331776
_CPT = _EPAD // (16 * 128)

_NC = 2
_NS = 16
_NW = _NC * _NS
_AR = 4096
_RNG = _AR - 1

_rng = np.random.default_rng(0)
_neg_all = _rng.integers(0, _N, size=(2, _E))
_neg_all = _neg_all[:, _neg_all[0] < _neg_all[1]]
_NUM_NEG = _neg_all.shape[1]
_NEGPAD = ((_NUM_NEG + 4095) // 4096) * 4096
_NCH = _NEGPAD // (_NW * 128)
_NPT = _NCH * 128
_neg0_np = np.zeros(_NEGPAD, np.int32)
_neg0_np[:_NUM_NEG] = _neg_all[0]
_neg1_np = np.zeros(_NEGPAD, np.int32)
_neg1_np[:_NUM_NEG] = _neg_all[1]
_negm_np = np.zeros(_NEGPAD, np.float32)
_negm_np[:_NUM_NEG] = 1.0

_mesh = plsc.VectorSubcoreMesh(core_axis_name="c", subcore_axis_name="s")
_f32 = jnp.float32
_i32 = jnp.int32

_ECH = _EPAD // (_NW * 128)
_EPT = _ECH * 128


def _route(didx_v, sidx_v, lidx_v, gsrc_v, lo, use_src):
    for g in range(8):
        d16 = didx_v[pl.ds(g * 16, 16)]
        ok = (d16 >= lo) & (d16 < lo + _RNG)
        lidx_v[pl.ds(g * 16, 16)] = jnp.where(ok, d16 - lo, _RNG)
        if use_src:
            s16 = sidx_v[pl.ds(g * 16, 16)]
            gsrc_v[pl.ds(g * 16, 16)] = jnp.where(ok, s16, 0)


@functools.partial(
    pl.kernel,
    out_type=jax.ShapeDtypeStruct((2 * _NC, _AR, 128), _f32),
    mesh=_mesh,
    compiler_params=pltpu.CompilerParams(needs_layout_passes=False),
    scratch_types=[
        pltpu.VMEM((128,), _i32),
        pltpu.VMEM((128,), _i32),
        pltpu.VMEM((128, 128), _f32),
        pltpu.VMEM((128, 128), _f32),
        pltpu.VMEM_SHARED((_AR, 128), _f32),
    ],
)
def _deg_kernel(dst_hbm, zer_hbm, out_hbm, didx_v, lidx_v, ones_v, zer_v,
                acc_sh):
    cid = lax.axis_index("c")
    sid = lax.axis_index("s")

    def _fill(i, _):
        for j in range(8):
            ones_v[i, pl.ds(j * 16, 16)] = jnp.ones((16,), _f32)
        return 0

    lax.fori_loop(0, 128, _fill, 0)
    pltpu.sync_copy(zer_hbm.at[pl.ds(0, 128)], zer_v)

    for p in range(2):
        lo = (2 * p + cid) * _RNG
        for q in range(2):
            pltpu.sync_copy(zer_v, acc_sh.at[pl.ds(sid * 256 + q * 128, 128)])
        plsc.subcore_barrier()

        def _step(k, _):
            base = pl.multiple_of(sid * (_CPT * 128) + k * 128, 128)
            pltpu.sync_copy(dst_hbm.at[pl.ds(base, 128)], didx_v)
            _route(didx_v, didx_v, lidx_v, lidx_v, lo, False)
            pltpu.sync_copy(ones_v, acc_sh.at[lidx_v], add=True)
            return 0

        lax.fori_loop(0, _CPT, _step, 0)
        plsc.subcore_barrier()
        pltpu.sync_copy(acc_sh.at[pl.ds(sid * 256, 256)],
                        out_hbm.at[2 * p + cid, pl.ds(sid * 256, 256)])
        plsc.subcore_barrier()


_NBUF = 3


@functools.partial(
    pl.kernel,
    out_type=jax.ShapeDtypeStruct((2 * _NC, _AR, 128), _f32),
    mesh=_mesh,
    compiler_params=pltpu.CompilerParams(needs_layout_passes=False),
    scratch_types=(
        [pltpu.VMEM((_CPT * 128,), _i32), pltpu.VMEM((_CPT * 128,), _i32)]
        + [pltpu.VMEM((128,), _i32) for _ in range(2 * _NBUF)]
        + [pltpu.VMEM((128, 128), _f32) for _ in range(_NBUF)]
        + [pltpu.VMEM_SHARED((_AR, 128), _f32)]
        + [pltpu.SemaphoreType.DMA for _ in range(2 * _NBUF)]
    ),
)
def _agg_kernel(g_hbm, src_hbm, dst_hbm, zer_hbm, out_hbm,
                sidx_all, didx_all, gi0, gi1, gi2, li0, li1, li2,
                r0, r1, r2, acc_sh, gs0, gs1, gs2, ss0, ss1, ss2):
    cid = lax.axis_index("c")
    sid = lax.axis_index("s")
    gidx = [gi0, gi1, gi2]
    lidx = [li0, li1, li2]
    rows = [r0, r1, r2]
    gsem = [gs0, gs1, gs2]
    ssem = [ss0, ss1, ss2]
    nblk = _CPT // _NBUF

    for p in range(2):
        lo = (2 * p + cid) * _RNG
        for q in range(2):
            pltpu.sync_copy(zer_hbm.at[pl.ds(0, 128)],
                            acc_sh.at[pl.ds(sid * 256 + q * 128, 128)])
        base_all = pl.multiple_of(sid * (_CPT * 128), 128)
        pltpu.sync_copy(src_hbm.at[pl.ds(base_all, _CPT * 128)], sidx_all)
        pltpu.sync_copy(dst_hbm.at[pl.ds(base_all, _CPT * 128)], didx_all)
        plsc.subcore_barrier()

        def _block(blk, _):
            for b in range(_NBUF):
                koff = blk * _NBUF + b

                @pl.when(blk > 0)
                def _drain_sc():
                    pltpu.make_async_copy(rows[b], acc_sh.at[lidx[b]],
                                          ssem[b]).wait()

                for g in range(8):
                    off = koff * 128 + g * 16
                    d16 = didx_all[pl.ds(off, 16)]
                    s16 = sidx_all[pl.ds(off, 16)]
                    ok = (d16 >= lo) & (d16 < lo + _RNG)
                    lidx[b][pl.ds(g * 16, 16)] = jnp.where(ok, d16 - lo, _RNG)
                    gidx[b][pl.ds(g * 16, 16)] = s16

                pltpu.async_copy(g_hbm.at[gidx[b]], rows[b], gsem[b])

                bp = (b - 1) % _NBUF
                if b == 0:
                    @pl.when(blk > 0)
                    def _fire_prev():
                        pltpu.make_async_copy(g_hbm.at[gidx[bp]], rows[bp],
                                              gsem[bp]).wait()
                        pltpu.async_copy(rows[bp], acc_sh.at[lidx[bp]],
                                         ssem[bp], add=True)
                else:
                    pltpu.make_async_copy(g_hbm.at[gidx[bp]], rows[bp],
                                          gsem[bp]).wait()
                    pltpu.async_copy(rows[bp], acc_sh.at[lidx[bp]],
                                     ssem[bp], add=True)
            return 0

        lax.fori_loop(0, nblk, _block, 0)
        pltpu.make_async_copy(g_hbm.at[gidx[2]], rows[2], gsem[2]).wait()
        pltpu.async_copy(rows[2], acc_sh.at[lidx[2]], ssem[2], add=True)
        for b in range(_NBUF):
            pltpu.make_async_copy(rows[b], acc_sh.at[lidx[b]], ssem[b]).wait()
        plsc.subcore_barrier()
        pltpu.sync_copy(acc_sh.at[pl.ds(sid * 256, 256)],
                        out_hbm.at[2 * p + cid, pl.ds(sid * 256, 256)])
        plsc.subcore_barrier()


@functools.partial(
    pl.kernel,
    out_type=jax.ShapeDtypeStruct((_NW, 128), _f32),
    mesh=_mesh,
    compiler_params=pltpu.CompilerParams(needs_layout_passes=False),
    scratch_types=[
        pltpu.VMEM((128,), _i32),
        pltpu.VMEM((128,), _i32),
        pltpu.VMEM((128,), _i32),
        pltpu.VMEM((128,), _f32),
        pltpu.VMEM((128,), _f32),
        pltpu.VMEM((128,), _f32),
        pltpu.VMEM((128, _H), _f32),
        pltpu.VMEM((128, _H), _f32),
        pltpu.SemaphoreType.DMA,
    ],
)
def _loss_kernel(rep_hbm, src_hbm, dst_hbm, simf_hbm, n0_hbm, n1_hbm, nm_hbm,
                 out_hbm, sidx_v, didx_v, qidx_v, fsim_v, nm_v, rowbuf,
                 rows_a, rows_b, sem):
    cid = lax.axis_index("c")
    sid = lax.axis_index("s")
    wid = cid * _NS + sid
    il = lax.iota(_i32, 16)

    def _dots16(g):
        rbase = il + g * 16

        def _dstep(dd, acc):
            cidx = jnp.zeros((16,), _i32) + dd
            ca = plsc.load_gather(rows_a, [rbase, cidx])
            cb = plsc.load_gather(rows_b, [rbase, cidx])
            return acc + ca * cb

        return lax.fori_loop(0, _H, _dstep, jnp.zeros((16,), _f32), unroll=8)

    def _pos_chunk(k, carry):
        ps, pc = carry
        base = pl.multiple_of(wid * _EPT + k * 128, 128)
        pltpu.sync_copy(src_hbm.at[pl.ds(base, 128)], sidx_v)
        pltpu.sync_copy(dst_hbm.at[pl.ds(base, 128)], didx_v)
        for j in range(8):
            s = sidx_v[pl.ds(j * 16, 16)]
            d = didx_v[pl.ds(j * 16, 16)]
            qidx_v[pl.ds(j * 16, 16)] = s * _N + d
        pltpu.async_copy(rep_hbm.at[sidx_v], rows_a, sem)
        pltpu.async_copy(rep_hbm.at[didx_v], rows_b, sem)
        pltpu.async_copy(simf_hbm.at[qidx_v], fsim_v, sem)
        pltpu.make_async_copy(rep_hbm.at[sidx_v], rows_a, sem).wait()
        pltpu.make_async_copy(rep_hbm.at[didx_v], rows_b, sem).wait()
        pltpu.make_async_copy(simf_hbm.at[qidx_v], fsim_v, sem).wait()
        for g in range(8):
            s16 = sidx_v[pl.ds(g * 16, 16)]
            d16 = didx_v[pl.ds(g * 16, 16)]
            fs16 = fsim_v[pl.ds(g * 16, 16)]
            w = jnp.maximum(_dots16(g), 0.0)
            pv = fs16 * _THETA + w * (1.0 - _THETA)
            valid = s16 < d16
            ps = ps + jnp.where(valid, (pv - 1.0) * (pv - 1.0), 0.0)
            pc = pc + jnp.where(valid, 1.0, 0.0)
        return (ps, pc)

    zz = jnp.zeros((16,), _f32)
    psv, pcv = lax.fori_loop(0, _ECH, _pos_chunk, (zz, zz))

    def _neg_chunk(k, carry):
        ns = carry
        base = pl.multiple_of(wid * _NPT + k * 128, 128)
        pltpu.sync_copy(n0_hbm.at[pl.ds(base, 128)], sidx_v)
        pltpu.sync_copy(n1_hbm.at[pl.ds(base, 128)], didx_v)
        pltpu.sync_copy(nm_hbm.at[pl.ds(base, 128)], nm_v)
        pltpu.async_copy(rep_hbm.at[sidx_v], rows_a, sem)
        pltpu.async_copy(rep_hbm.at[didx_v], rows_b, sem)
        pltpu.make_async_copy(rep_hbm.at[sidx_v], rows_a, sem).wait()
        pltpu.make_async_copy(rep_hbm.at[didx_v], rows_b, sem).wait()
        for g in range(8):
            nm16 = nm_v[pl.ds(g * 16, 16)]
            w = jnp.maximum(_dots16(g), 0.0)
            ns = ns + nm16 * w * w
        return ns

    nsv = lax.fori_loop(0, _NCH, _neg_chunk, zz)
    psum = jnp.sum(psv)
    pcnt = jnp.sum(pcv)
    nsum = jnp.sum(nsv)

    for j in range(8):
        rowbuf[pl.ds(j * 16, 16)] = jnp.zeros((16,), _f32)
    vec = (jnp.where(il == 0, psum, 0.0) + jnp.where(il == 1, pcnt, 0.0)
           + jnp.where(il == 2, nsum, 0.0))
    rowbuf[pl.ds(0, 16)] = vec
    pltpu.sync_copy(rowbuf, out_hbm.at[wid])


_BLK = 512
_GRID = _NPAD // _BLK


def _mm1_body(deg_ref, feat_ref, w1_ref, dis_ref, g1_ref):
    i = pl.program_id(0)
    dis = lax.rsqrt(deg_ref[pl.ds(i * _BLK, _BLK)])
    dis_ref[pl.ds(i * _BLK, _BLK)] = dis
    h = lax.dot_general(feat_ref[...], w1_ref[...], (((1,), (1,)), ((), ())),
                        preferred_element_type=_f32)
    g1_ref[...] = dis[:, None] * h


def _mm1(deg, feat_p, W1):
    return pl.pallas_call(
        _mm1_body,
        grid=(_GRID,),
        in_specs=[
            pl.BlockSpec((_NPAD,), lambda i: (0,)),
            pl.BlockSpec((_BLK, _D), lambda i: (i, 0)),
            pl.BlockSpec((_H, _D), lambda i: (0, 0)),
        ],
        out_specs=[
            pl.BlockSpec((_NPAD,), lambda i: (0,)),
            pl.BlockSpec((_BLK, _H), lambda i: (i, 0)),
        ],
        out_shape=[
            jax.ShapeDtypeStruct((_NPAD,), _f32),
            jax.ShapeDtypeStruct((_NPAD, _H), _f32),
        ],
    )(deg, feat_p, W1)


def _mid_body(dis_ref, b1_ref, agg_ref, g1_ref, w2_ref, g2_ref):
    i = pl.program_id(0)
    dis = dis_ref[pl.ds(i * _BLK, _BLK)]
    agg = agg_ref[...] + g1_ref[...]
    x1 = jnp.maximum(dis[:, None] * agg + b1_ref[...][None, :], 0.0)
    h2 = lax.dot_general(x1, w2_ref[...], (((1,), (1,)), ((), ())),
                         preferred_element_type=_f32)
    g2_ref[...] = dis[:, None] * h2


def _mid(dis, b1, agg1, g1, W2):
    return pl.pallas_call(
        _mid_body,
        grid=(_GRID,),
        in_specs=[
            pl.BlockSpec((_NPAD,), lambda i: (0,)),
            pl.BlockSpec((_H,), lambda i: (0,)),
            pl.BlockSpec((_BLK, _H), lambda i: (i, 0)),
            pl.BlockSpec((_BLK, _H), lambda i: (i, 0)),
            pl.BlockSpec((_H, _H), lambda i: (0, 0)),
        ],
        out_specs=pl.BlockSpec((_BLK, _H), lambda i: (i, 0)),
        out_shape=jax.ShapeDtypeStruct((_NPAD, _H), _f32),
    )(dis, b1, agg1, g1, W2)


def _head_body(dis_ref, b2_ref, by_ref, agg_ref, g2_ref, wy_ref,
               rep_ref, y_ref):
    i = pl.program_id(0)
    dis = dis_ref[pl.ds(i * _BLK, _BLK)]
    agg = agg_ref[...] + g2_ref[...]
    x2 = dis[:, None] * agg + b2_ref[...][None, :]
    n1 = jnp.sqrt(jnp.sum(x2 * x2, axis=1, keepdims=True))
    r1 = x2 / jnp.maximum(n1, 1e-12)
    n2 = jnp.sqrt(jnp.sum(r1 * r1, axis=1, keepdims=True))
    rep = r1 / jnp.maximum(n2, 1e-12)
    rep_ref[...] = rep
    y_ref[...] = lax.dot_general(rep, wy_ref[...], (((1,), (1,)), ((), ())),
                                 preferred_element_type=_f32) + by_ref[...][None, :]


def _head(dis, b2, by_p, agg2, g2, Wy_p):
    return pl.pallas_call(
        _head_body,
        grid=(_GRID,),
        in_specs=[
            pl.BlockSpec((_NPAD,), lambda i: (0,)),
            pl.BlockSpec((_H,), lambda i: (0,)),
            pl.BlockSpec((128,), lambda i: (0,)),
            pl.BlockSpec((_BLK, _H), lambda i: (i, 0)),
            pl.BlockSpec((_BLK, _H), lambda i: (i, 0)),
            pl.BlockSpec((128, _H), lambda i: (0, 0)),
        ],
        out_specs=[
            pl.BlockSpec((_BLK, _H), lambda i: (i, 0)),
            pl.BlockSpec((_BLK, 128), lambda i: (i, 0)),
        ],
        out_shape=[
            jax.ShapeDtypeStruct((_NPAD, _H), _f32),
            jax.ShapeDtypeStruct((_NPAD, 128), _f32),
        ],
    )(dis, b2, by_p, agg2, g2, Wy_p)


def _fin_body(parts_ref, out_ref):
    parts = parts_ref[...]
    col = lax.broadcasted_iota(_i32, (_NW, 128), 1)
    ps = jnp.sum(jnp.where(col == 0, parts, 0.0))
    pc = jnp.sum(jnp.where(col == 1, parts, 0.0))
    ns = jnp.sum(jnp.where(col == 2, parts, 0.0))
    rec = (ns + ps) * float(_N) / (float(_NUM_NEG) + pc)
    out_ref[...] = jnp.full((8, 128), rec, _f32)


def _fin(parts):
    return pl.pallas_call(
        _fin_body,
        grid=(1,),
        in_specs=[pl.BlockSpec((_NW, 128), lambda i: (0, 0))],
        out_specs=pl.BlockSpec((8, 128), lambda i: (0, 0)),
        out_shape=jax.ShapeDtypeStruct((8, 128), _f32),
    )(parts)


def _assemble(outp):
    return jnp.concatenate(
        [outp[0, :_RNG], outp[1, :_RNG], outp[2, :_NPAD - 2 * _RNG]], axis=0)


def kernel(edge_index, features, sim, W1, b1, W2, b2, Wy, by):
    src = edge_index[0].astype(_i32)
    dst = edge_index[1].astype(_i32)
    padz = jnp.zeros(_EPAD - _E, _i32)
    src_p = jnp.concatenate([src, padz])
    dst_l = jnp.concatenate([dst, padz])
    dst_a = jnp.concatenate([dst, jnp.full(_EPAD - _E, _N, _i32)])

    feat_p = jnp.pad(features, ((0, _NPAD - _N), (0, 0)))
    Wy_p = jnp.pad(Wy, ((0, 128 - _C), (0, 0)))
    by_p = jnp.pad(by, (0, 128 - _C))
    sim_flat = sim.reshape(-1)
    zer128 = jnp.zeros((128, 128), _f32)

    neg0 = jnp.asarray(_neg0_np)
    neg1 = jnp.asarray(_neg1_np)
    negm = jnp.asarray(_negm_np)

    degp = _deg_kernel(dst_a, zer128)
    deg = _assemble(degp)[:, 0] + 1.0

    dis, g1 = _mm1(deg, feat_p, W1)
    agg1 = _assemble(_agg_kernel(g1, src_p, dst_a, zer128))
    g2 = _mid(dis, b1, agg1, g1, W2)
    agg2 = _assemble(_agg_kernel(g2, src_p, dst_a, zer128))
    rep_p, y_p = _head(dis, b2, by_p, agg2, g2, Wy_p)

    parts = _loss_kernel(rep_p, src_p, dst_l, sim_flat, neg0, neg1, negm)
    recb = _fin(parts)

    return rep_p[:_N], recb[0, 0], y_p[:_N, :_C]
